# baseline (speedup 1.0000x reference)
import jax
import jax.numpy as jnp
from jax import lax
from jax.experimental import pallas as pl
from jax.experimental.pallas import tpu as pltpu
from jax.experimental.pallas import tpu_sc as plsc

H = 32
W = 32
D = 128
BS = 16
NC = 2
NS = 16
L = 16


def _pos_body(row_hbm, col_hbm, out_hbm, tabr, pl0, pl1, sem):
    cid = lax.axis_index("c")
    sid = lax.axis_index("s")

    pltpu.sync_copy(row_hbm.at[pl.ds(0, H)], tabr)

    pltpu.sync_copy(col_hbm.at[pl.ds(0, H)], pl0.at[:, pl.ds(0, D)])
    pltpu.sync_copy(col_hbm.at[pl.ds(0, H)], pl1.at[:, pl.ds(0, D)])

    for k in range(D // L):
        v0 = tabr[sid, pl.ds(k * L, L)]
        v1 = tabr[sid + NS, pl.ds(k * L, L)]
        for w in range(W):
            pl0[w, pl.ds(D + k * L, L)] = v0
            pl1[w, pl.ds(D + k * L, L)] = v1

    copies = []
    for b in range(BS // NC):
        bb = cid * (BS // NC) + b
        copies.append(pltpu.async_copy(pl0, out_hbm.at[bb, sid], sem))
        copies.append(pltpu.async_copy(pl1, out_hbm.at[bb, sid + NS], sem))
    for c in copies:
        c.wait()


@jax.jit
def _pos_embed(row_embed, col_embed):
    mesh = plsc.VectorSubcoreMesh(
        core_axis_name="c", subcore_axis_name="s", num_cores=NC, num_subcores=NS
    )
    out = pl.kernel(
        _pos_body,
        out_type=jax.ShapeDtypeStruct((BS, H, W, 2 * D), jnp.float32),
        mesh=mesh,
        scratch_types=[
            pltpu.VMEM((H, D), jnp.float32),
            pltpu.VMEM((W, 2 * D), jnp.float32),
            pltpu.VMEM((W, 2 * D), jnp.float32),
            pltpu.SemaphoreType.DMA,
        ],
        compiler_params=pltpu.CompilerParams(
            needs_layout_passes=False, use_tc_tiling_on_sc=True
        ),
    )(row_embed, col_embed)
    return jnp.transpose(out, (0, 3, 1, 2))


def kernel(mask, row_embed, col_embed):
    del mask
    return _pos_embed(row_embed, col_embed)


# speedup vs baseline: 5.5415x; 5.5415x over previous
import jax
import jax.numpy as jnp
from jax.experimental import pallas as pl
from jax.experimental.pallas import tpu as pltpu

H = 32
W = 32
D = 128
BS = 16


def _body(row_ref, col_ref, out_ref, img, sems):
    col32 = col_ref[...]
    row32 = row_ref[...]
    colB = jnp.broadcast_to(col32[None, :, :], (H, W, D))
    rowB = jnp.broadcast_to(row32[:, None, :], (H, W, D))
    img[...] = jnp.concatenate([colB, rowB], axis=-1)

    copies = [
        pltpu.make_async_copy(img, out_ref.at[b], sems.at[b]) for b in range(BS)
    ]
    for c in copies:
        c.start()
    for c in copies:
        c.wait()


@jax.jit
def _pos_embed(row_embed, col_embed):
    out = pl.pallas_call(
        _body,
        grid=(1,),
        in_specs=[
            pl.BlockSpec((H, D), lambda i: (0, 0)),
            pl.BlockSpec((H, D), lambda i: (0, 0)),
        ],
        out_specs=pl.BlockSpec(memory_space=pl.ANY),
        out_shape=jax.ShapeDtypeStruct((BS, H, W, 2 * D), jnp.float32),
        scratch_shapes=[
            pltpu.VMEM((H, W, 2 * D), jnp.float32),
            pltpu.SemaphoreType.DMA((BS,)),
        ],
    )(row_embed, col_embed)
    return jnp.transpose(out, (0, 3, 1, 2))


def kernel(mask, row_embed, col_embed):
    del mask
    return _pos_embed(row_embed, col_embed)
